# baseline (speedup 1.0000x reference)
import functools

import jax
import jax.numpy as jnp
from jax import lax
from jax.experimental import pallas as pl
from jax.experimental.pallas import tpu as pltpu
from jax.experimental.pallas import tpu_sc as plsc

_NC = 2
_NS = 16
_NW = _NC * _NS
_LANES = 16
_NODES_PER_CHUNK = 4
_NBUF = 4


def _sc_neighbor_sums(tn, feat, S):
    nw, nch, rows_per_chunk = tn.shape
    _, D = feat.shape
    npc = rows_per_chunk // S
    cpw = nch * npc
    B = nw * cpw
    dv = D // _LANES
    mesh = plsc.VectorSubcoreMesh(
        core_axis_name="c", subcore_axis_name="s",
        num_cores=_NC, num_subcores=_NS)

    @functools.partial(
        pl.kernel,
        out_type=jax.ShapeDtypeStruct((B, D), jnp.float32),
        mesh=mesh,
        scratch_types=[
            pltpu.VMEM((nch, rows_per_chunk), jnp.int32),
            pltpu.VMEM((_NBUF, rows_per_chunk, D), jnp.float32),
            pltpu.VMEM((2, npc, D), jnp.float32),
            [pltpu.SemaphoreType.DMA] * _NBUF,
            [pltpu.SemaphoreType.DMA] * 2,
        ],
    )
    def sums_kernel(tn_hbm, feat_hbm, out_hbm, idx_v, rows_v, out_s,
                    sems, semo):
        wid = lax.axis_index("s") * _NC + lax.axis_index("c")
        pltpu.sync_copy(tn_hbm.at[wid], idx_v)
        for k in range(_NBUF):
            pltpu.async_copy(feat_hbm.at[idx_v.at[k]], rows_v.at[k], sems[k])

        def group(gc, carry):
            c0 = _NBUF * gc
            for k in range(_NBUF):
                c = c0 + k
                ko = k % 2
                pltpu.make_async_copy(
                    feat_hbm.at[idx_v.at[k]], rows_v.at[k], sems[k]).wait()

                @pl.when(c >= 2)
                def _():
                    pltpu.make_async_copy(
                        out_s.at[ko], out_hbm.at[pl.ds(0, npc)],
                        semo[ko]).wait()

                buf = rows_v.at[k]
                for j in range(npc):
                    accs = [buf[j * S, pl.ds(d * _LANES, _LANES)]
                            for d in range(dv)]
                    for s in range(1, S):
                        for d in range(dv):
                            accs[d] = accs[d] + buf[
                                j * S + s, pl.ds(d * _LANES, _LANES)]
                    for d in range(dv):
                        out_s[ko, j, pl.ds(d * _LANES, _LANES)] = accs[d]
                pltpu.async_copy(
                    out_s.at[ko],
                    out_hbm.at[pl.ds(wid * cpw + c * npc, npc)], semo[ko])

                @pl.when(c + _NBUF < nch)
                def _():
                    pltpu.async_copy(
                        feat_hbm.at[idx_v.at[c + _NBUF]], rows_v.at[k],
                        sems[k])
            return carry

        lax.fori_loop(0, nch // _NBUF, group, 0)
        for ko in range(2):
            pltpu.make_async_copy(
                out_s.at[ko], out_hbm.at[pl.ds(0, npc)], semo[ko]).wait()

    return sums_kernel(tn, feat)


def _tc_linear(x, W, b, S):
    B, D_IN = x.shape
    D_OUT = W.shape[1]
    blk = min(B, 2048)
    scale = 1.0 / S

    def body(x_ref, w_ref, b_ref, o_ref):
        o_ref[...] = (
            jnp.dot(x_ref[...], w_ref[...], preferred_element_type=jnp.float32)
            * scale + b_ref[...])

    return pl.pallas_call(
        body,
        grid=(B // blk,),
        in_specs=[
            pl.BlockSpec((blk, D_IN), lambda i: (i, 0)),
            pl.BlockSpec((D_IN, D_OUT), lambda i: (0, 0)),
            pl.BlockSpec((1, D_OUT), lambda i: (0, 0)),
        ],
        out_specs=pl.BlockSpec((blk, D_OUT), lambda i: (i, 0)),
        out_shape=jax.ShapeDtypeStruct((B, D_OUT), jnp.float32),
    )(x, W, b.reshape(1, D_OUT))


def kernel(nodes, to_neighs, id2feat, W, b):
    B, S = to_neighs.shape
    rows_per_chunk = _NODES_PER_CHUNK * S
    nch = B // (_NW * _NODES_PER_CHUNK)
    tn = to_neighs.astype(jnp.int32).reshape(_NW, nch, rows_per_chunk)
    sums = _sc_neighbor_sums(tn, id2feat, S)
    return _tc_linear(sums, W, b, S)


# speedup vs baseline: 3.0065x; 3.0065x over previous
import functools

import jax
import jax.numpy as jnp
from jax import lax
from jax.experimental import pallas as pl
from jax.experimental.pallas import tpu as pltpu
from jax.experimental.pallas import tpu_sc as plsc

_NC = 2
_NS = 16
_NW = _NC * _NS
_LANES = 16
_NODES_PER_CHUNK = 4
_NBUF = 4


def _sc_neighbor_sums(tn, feat, S):
    nw, nch, rows_per_chunk = tn.shape
    _, D = feat.shape
    npc = rows_per_chunk // S
    cpw = nch * npc
    B = nw * cpw
    dv = D // _LANES
    mesh = plsc.VectorSubcoreMesh(
        core_axis_name="c", subcore_axis_name="s",
        num_cores=_NC, num_subcores=_NS)

    @functools.partial(
        pl.kernel,
        out_type=jax.ShapeDtypeStruct((B, D), jnp.float32),
        mesh=mesh,
        scratch_types=[
            pltpu.VMEM((nch, rows_per_chunk), jnp.int32),
            pltpu.VMEM((_NBUF, rows_per_chunk, D), jnp.float32),
            pltpu.VMEM((2, npc, D), jnp.float32),
            [pltpu.SemaphoreType.DMA] * _NBUF,
            [pltpu.SemaphoreType.DMA] * 2,
        ],
    )
    def sums_kernel(tn_hbm, feat_hbm, out_hbm, idx_v, rows_v, out_s,
                    sems, semo):
        wid = lax.axis_index("s") * _NC + lax.axis_index("c")
        pltpu.sync_copy(tn_hbm.at[wid], idx_v)
        for k in range(_NBUF):
            pltpu.async_copy(feat_hbm.at[idx_v.at[k]], rows_v.at[k], sems[k])

        def group(gc, carry):
            c0 = _NBUF * gc
            for k in range(_NBUF):
                c = c0 + k
                ko = k % 2
                pltpu.make_async_copy(
                    feat_hbm.at[idx_v.at[k]], rows_v.at[k], sems[k]).wait()

                @pl.when(c >= 2)
                def _():
                    pltpu.make_async_copy(
                        out_s.at[ko], out_hbm.at[pl.ds(0, npc)],
                        semo[ko]).wait()

                buf = rows_v.at[k]
                for j in range(npc):
                    def body(s, accs):
                        return tuple(
                            accs[d] + buf[j * S + s, pl.ds(d * _LANES, _LANES)]
                            for d in range(dv))
                    accs = lax.fori_loop(
                        0, S, body,
                        tuple(jnp.zeros((_LANES,), jnp.float32)
                              for _ in range(dv)),
                        unroll=8)
                    for d in range(dv):
                        out_s[ko, j, pl.ds(d * _LANES, _LANES)] = accs[d]
                pltpu.async_copy(
                    out_s.at[ko],
                    out_hbm.at[pl.ds(wid * cpw + c * npc, npc)], semo[ko])

                @pl.when(c + _NBUF < nch)
                def _():
                    pltpu.async_copy(
                        feat_hbm.at[idx_v.at[c + _NBUF]], rows_v.at[k],
                        sems[k])
            return carry

        lax.fori_loop(0, nch // _NBUF, group, 0)
        for ko in range(2):
            pltpu.make_async_copy(
                out_s.at[ko], out_hbm.at[pl.ds(0, npc)], semo[ko]).wait()

    return sums_kernel(tn, feat)


def _tc_linear(x, W, b, S):
    B, D_IN = x.shape
    D_OUT = W.shape[1]
    blk = min(B, 2048)
    scale = 1.0 / S

    def body(x_ref, w_ref, b_ref, o_ref):
        o_ref[...] = (
            jnp.dot(x_ref[...], w_ref[...], preferred_element_type=jnp.float32)
            * scale + b_ref[...])

    return pl.pallas_call(
        body,
        grid=(B // blk,),
        in_specs=[
            pl.BlockSpec((blk, D_IN), lambda i: (i, 0)),
            pl.BlockSpec((D_IN, D_OUT), lambda i: (0, 0)),
            pl.BlockSpec((1, D_OUT), lambda i: (0, 0)),
        ],
        out_specs=pl.BlockSpec((blk, D_OUT), lambda i: (i, 0)),
        out_shape=jax.ShapeDtypeStruct((B, D_OUT), jnp.float32),
    )(x, W, b.reshape(1, D_OUT))


def kernel(nodes, to_neighs, id2feat, W, b):
    B, S = to_neighs.shape
    rows_per_chunk = _NODES_PER_CHUNK * S
    nch = B // (_NW * _NODES_PER_CHUNK)
    tn = to_neighs.astype(jnp.int32).reshape(_NW, nch, rows_per_chunk)
    sums = _sc_neighbor_sums(tn, id2feat, S)
    return _tc_linear(sums, W, b, S)
